# baseline (speedup 1.0000x reference)
import functools

import jax
import jax.numpy as jnp
from jax import lax
from jax.experimental import pallas as pl
from jax.experimental.pallas import tpu as pltpu
from jax.experimental.pallas import tpu_sc as plsc

NCORES = 2
NSUB = 16
NW = NCORES * NSUB
CHUNK = 128
NBUF = 4

_F32_MAX = float(jnp.finfo(jnp.float32).max)
_F32_MIN = float(jnp.finfo(jnp.float32).min)


def _seg_body(n_iter, rps, x_hbm, src_hbm, dst_hbm, zero_hbm,
              aggp_hbm, src_v, dst_v, rows_v, acc_sh, sem):
    c = lax.axis_index("c")
    s = lax.axis_index("s")
    w = s * NCORES + c
    pltpu.sync_copy(src_hbm.at[w], src_v)
    pltpu.sync_copy(dst_hbm.at[w], dst_v)
    r0 = s * rps
    pltpu.sync_copy(zero_hbm.at[pl.ds(r0, rps)], acc_sh.at[pl.ds(r0, rps)])
    plsc.subcore_barrier()

    def step(i, carry):
        pltpu.async_copy(x_hbm.at[src_v.at[i]], rows_v, sem).wait()
        pltpu.sync_copy(rows_v, acc_sh.at[dst_v.at[i]], add=True)
        return carry

    lax.fori_loop(0, n_iter, step, 0)
    plsc.subcore_barrier()
    pltpu.sync_copy(acc_sh.at[pl.ds(r0, rps)],
                    aggp_hbm.at[c].at[pl.ds(r0, rps)])


def _make_seg_sum(n, n_iter, d):
    rps = n // NSUB
    mesh = plsc.VectorSubcoreMesh(core_axis_name="c", subcore_axis_name="s")
    return pl.kernel(
        functools.partial(_seg_body, n_iter, rps),
        out_type=jax.ShapeDtypeStruct((NCORES, n, d), jnp.float32),
        mesh=mesh,
        scratch_types=[
            pltpu.VMEM((n_iter, CHUNK), jnp.int32),
            pltpu.VMEM((n_iter, CHUNK), jnp.int32),
            pltpu.VMEM((CHUNK, d), jnp.float32),
            pltpu.VMEM_SHARED((n, d), jnp.float32),
            pltpu.SemaphoreType.DMA,
        ],
    )


def _cnt_body(n_iter, rps, dst_hbm, zero_hbm, one_hbm, cntp_hbm,
              dst_v, ones_v, cnt_sh, sem):
    c = lax.axis_index("c")
    s = lax.axis_index("s")
    w = s * NCORES + c
    pltpu.sync_copy(dst_hbm.at[w], dst_v)
    pltpu.sync_copy(one_hbm, ones_v)
    r0 = s * rps
    pltpu.sync_copy(zero_hbm.at[pl.ds(r0, rps)], cnt_sh.at[pl.ds(r0, rps)])
    plsc.subcore_barrier()

    def super_step(j, carry):
        i0 = j * NBUF
        sd = [pltpu.async_copy(ones_v, cnt_sh.at[dst_v.at[i0 + b]], sem,
                               add=True) for b in range(NBUF)]
        for b in range(NBUF):
            sd[b].wait()
        return carry

    lax.fori_loop(0, n_iter // NBUF, super_step, 0)
    plsc.subcore_barrier()
    pltpu.sync_copy(cnt_sh.at[pl.ds(r0, rps)],
                    cntp_hbm.at[c].at[pl.ds(r0, rps)])


def _make_cnt(n, n_iter, d):
    rps = n // NSUB
    mesh = plsc.VectorSubcoreMesh(core_axis_name="c", subcore_axis_name="s")
    return pl.kernel(
        functools.partial(_cnt_body, n_iter, rps),
        out_type=jax.ShapeDtypeStruct((NCORES, n, d), jnp.float32),
        mesh=mesh,
        scratch_types=[
            pltpu.VMEM((n_iter, CHUNK), jnp.int32),
            pltpu.VMEM((CHUNK, d), jnp.float32),
            pltpu.VMEM_SHARED((n, d), jnp.float32),
            pltpu.SemaphoreType.DMA,
        ],
    )


def _gather_body(rows_per_w, n_stream, h_hbm, idx_hbm, out_hbm,
                 idx_v, rows_v, sem):
    c = lax.axis_index("c")
    s = lax.axis_index("s")
    w = s * NCORES + c
    pltpu.sync_copy(idx_hbm.at[w], idx_v)
    for j in range(rows_per_w // n_stream):
        pltpu.async_copy(h_hbm.at[idx_v.at[j]], rows_v, sem).wait()
        pltpu.sync_copy(
            rows_v, out_hbm.at[pl.ds(w * rows_per_w + j * n_stream, n_stream)])


def _make_gather(n, d, b):
    rows_per_w = b // NW
    n_stream = 128
    mesh = plsc.VectorSubcoreMesh(core_axis_name="c", subcore_axis_name="s")
    return pl.kernel(
        functools.partial(_gather_body, rows_per_w, n_stream),
        out_type=jax.ShapeDtypeStruct((b, d), jnp.float32),
        mesh=mesh,
        scratch_types=[
            pltpu.VMEM((rows_per_w // n_stream, n_stream), jnp.int32),
            pltpu.VMEM((n_stream, d), jnp.float32),
            pltpu.SemaphoreType.DMA,
        ],
    )


def _dot_t(a, w):
    return lax.dot_general(a, w, (((1,), (1,)), ((), ())),
                           preferred_element_type=jnp.float32)


def _dense_body(final, n, p_ref, c_ref, h_ref, wl_ref, wr_ref, b_ref,
                g_ref, bt_ref, o_ref):
    agg = p_ref[0, 0:n] + p_ref[1, 0:n]
    cnt = c_ref[0, 0:n, 0:1] + c_ref[1, 0:n, 0:1]
    mean = agg / jnp.maximum(cnt, 1.0)
    t = (_dot_t(mean, wl_ref[...]) + _dot_t(h_ref[...], wr_ref[...])
         + b_ref[...])
    mu = jnp.mean(t, axis=0, keepdims=True)
    var = jnp.mean((t - mu) ** 2, axis=0, keepdims=True)
    hn = g_ref[...] * (t - mu) * lax.rsqrt(var + 1e-5) + bt_ref[...]
    hn = jnp.where(hn >= 0, hn, 0.01 * hn)
    if final:
        hn = jnp.where(jnp.isnan(hn), jnp.float32(1e-14), hn)
        hn = jnp.clip(hn, _F32_MIN, _F32_MAX)
    o_ref[...] = hn


def _make_dense(n, h, final):
    return pl.pallas_call(
        functools.partial(_dense_body, final, n),
        out_shape=jax.ShapeDtypeStruct((n, h), jnp.float32),
    )


def _mlp_body(nc, rows_ref, feat_ref, w0u_ref, w0v_ref, w0f_ref, b0_ref,
              w1_ref, b1_ref, w2_ref, b2_ref, y_ref, p_ref):
    hu = rows_ref[0:nc, :]
    hv = rows_ref[nc:2 * nc, :]
    z = (_dot_t(hu, w0u_ref[...]) + _dot_t(hv, w0v_ref[...])
         + feat_ref[...] * w0f_ref[...] + b0_ref[...])
    z = jnp.where(z >= 0, z, 0.01 * z)
    z = _dot_t(z, w1_ref[...]) + b1_ref[...]
    z = jnp.where(z >= 0, z, 0.01 * z)
    y = jnp.sum(z * w2_ref[...], axis=1, keepdims=True) + b2_ref[0, 0]
    m = jnp.max(y)
    ey = jnp.exp(y - m)
    p_ref[...] = ey / jnp.sum(ey)
    y_ref[...] = y


def _make_mlp(nc):
    return pl.pallas_call(
        functools.partial(_mlp_body, nc),
        out_shape=(jax.ShapeDtypeStruct((nc, 1), jnp.float32),
                   jax.ShapeDtypeStruct((nc, 1), jnp.float32)),
    )


def kernel(x, edge_index, cand_uv, cand_feat,
           conv0_wl, conv0_wr, conv0_b,
           conv1_wl, conv1_wr, conv1_b,
           bn0_g, bn0_b, bn1_g, bn1_b,
           mlp_w0, mlp_b0, mlp_w1, mlp_b1, mlp_w2, mlp_b2):
    n, d = x.shape
    e = edge_index.shape[1]
    nc = cand_uv.shape[0]
    h = conv0_wl.shape[0]
    grp = NW * CHUNK * NBUF
    epad = ((e + grp - 1) // grp) * grp
    n_iter = epad // (NW * CHUNK)

    npad = ((n + 8 * NSUB - 1) // (8 * NSUB)) * (8 * NSUB)
    src_flat = jnp.concatenate(
        [edge_index[0].astype(jnp.int32),
         jnp.zeros((epad - e,), jnp.int32)])
    dst_flat = jnp.concatenate(
        [edge_index[1].astype(jnp.int32),
         jnp.full((epad - e,), n, jnp.int32)])
    src3 = src_flat.reshape(NW, n_iter, CHUNK)
    dst3 = dst_flat.reshape(NW, n_iter, CHUNK)
    z128 = jnp.zeros((npad, d), jnp.float32)
    ones = jnp.ones((CHUNK, d), jnp.float32)
    cidx = jnp.concatenate(
        [cand_uv[:, 0], cand_uv[:, 1]]).astype(jnp.int32).reshape(NW, -1, 128)

    seg = _make_seg_sum(npad, n_iter, d)
    cntk = _make_cnt(npad, n_iter, d)
    gat = _make_gather(n, h, 2 * nc)
    dense0 = _make_dense(n, h, False)
    dense1 = _make_dense(n, h, True)
    mlp = _make_mlp(nc)

    cntp = cntk(dst3, z128, ones)
    aggp0 = seg(x, src3, dst3, z128)
    h1 = dense0(aggp0, cntp, x, conv0_wl, conv0_wr,
                conv0_b.reshape(1, h), bn0_g.reshape(1, h),
                bn0_b.reshape(1, h))
    aggp1 = seg(h1, src3, dst3, z128)
    h2 = dense1(aggp1, cntp, h1, conv1_wl, conv1_wr,
                conv1_b.reshape(1, h), bn1_g.reshape(1, h),
                bn1_b.reshape(1, h))
    rows = gat(h2, cidx)
    y, p = mlp(rows, cand_feat,
               mlp_w0[:, 0:d], mlp_w0[:, d:2 * d],
               mlp_w0[:, 2 * d].reshape(1, -1), mlp_b0.reshape(1, -1),
               mlp_w1, mlp_b1.reshape(1, -1),
               mlp_w2, mlp_b2.reshape(1, 1))
    return (y, p)


# speedup vs baseline: 2.3655x; 2.3655x over previous
import functools

import jax
import jax.numpy as jnp
from jax import lax
from jax.experimental import pallas as pl
from jax.experimental.pallas import tpu as pltpu
from jax.experimental.pallas import tpu_sc as plsc

NCORES = 2
NSUB = 16
NW = NCORES * NSUB
CHUNK = 125
NBUF = 4

_F32_MAX = float(jnp.finfo(jnp.float32).max)
_F32_MIN = float(jnp.finfo(jnp.float32).min)


def _seg_body(n_iter, rps, x_hbm, src_hbm, dst_hbm, zero_hbm,
              aggp_hbm, src_v, dst_v, rows_v, acc_sh, sem):
    c = lax.axis_index("c")
    s = lax.axis_index("s")
    w = s * NCORES + c
    pltpu.sync_copy(src_hbm.at[w], src_v)
    pltpu.sync_copy(dst_hbm.at[w], dst_v)
    r0 = s * rps
    pltpu.sync_copy(zero_hbm.at[pl.ds(r0, rps)], acc_sh.at[pl.ds(r0, rps)])
    plsc.subcore_barrier()

    def step(i, carry):
        pltpu.async_copy(x_hbm.at[src_v.at[i]], rows_v, sem).wait()
        pltpu.sync_copy(rows_v, acc_sh.at[dst_v.at[i]], add=True)
        return carry

    lax.fori_loop(0, n_iter, step, 0)
    plsc.subcore_barrier()
    pltpu.sync_copy(acc_sh.at[pl.ds(r0, rps)],
                    aggp_hbm.at[c].at[pl.ds(r0, rps)])


def _make_seg_sum(n, n_iter, d):
    rps = n // NSUB
    mesh = plsc.VectorSubcoreMesh(core_axis_name="c", subcore_axis_name="s")
    return pl.kernel(
        functools.partial(_seg_body, n_iter, rps),
        out_type=jax.ShapeDtypeStruct((NCORES, n, d), jnp.float32),
        mesh=mesh,
        scratch_types=[
            pltpu.VMEM((n_iter, CHUNK), jnp.int32),
            pltpu.VMEM((n_iter, CHUNK), jnp.int32),
            pltpu.VMEM((CHUNK, d), jnp.float32),
            pltpu.VMEM_SHARED((n, d), jnp.float32),
            pltpu.SemaphoreType.DMA,
        ],
    )


def _cnt_body(n_iter, rps, dst_hbm, zero_hbm, one_hbm, cntp_hbm,
              dst_v, ones_v, cnt_sh, sem):
    c = lax.axis_index("c")
    s = lax.axis_index("s")
    w = s * NCORES + c
    pltpu.sync_copy(dst_hbm.at[w], dst_v)
    pltpu.sync_copy(one_hbm, ones_v)
    r0 = s * rps
    pltpu.sync_copy(zero_hbm.at[pl.ds(r0, rps)], cnt_sh.at[pl.ds(r0, rps)])
    plsc.subcore_barrier()

    def super_step(j, carry):
        i0 = j * NBUF
        sd = [pltpu.async_copy(ones_v, cnt_sh.at[dst_v.at[i0 + b]], sem,
                               add=True) for b in range(NBUF)]
        for b in range(NBUF):
            sd[b].wait()
        return carry

    lax.fori_loop(0, n_iter // NBUF, super_step, 0)
    plsc.subcore_barrier()
    pltpu.sync_copy(cnt_sh.at[pl.ds(r0, rps)],
                    cntp_hbm.at[c].at[pl.ds(r0, rps)])


def _make_cnt(n, n_iter, d):
    rps = n // NSUB
    mesh = plsc.VectorSubcoreMesh(core_axis_name="c", subcore_axis_name="s")
    return pl.kernel(
        functools.partial(_cnt_body, n_iter, rps),
        out_type=jax.ShapeDtypeStruct((NCORES, n, d), jnp.float32),
        mesh=mesh,
        scratch_types=[
            pltpu.VMEM((n_iter, CHUNK), jnp.int32),
            pltpu.VMEM((CHUNK, d), jnp.float32),
            pltpu.VMEM_SHARED((n, d), jnp.float32),
            pltpu.SemaphoreType.DMA,
        ],
    )


def _gather_body(rows_per_w, n_stream, h_hbm, idx_hbm, out_hbm,
                 idx_v, rows_v, sem):
    c = lax.axis_index("c")
    s = lax.axis_index("s")
    w = s * NCORES + c
    pltpu.sync_copy(idx_hbm.at[w], idx_v)
    for j in range(rows_per_w // n_stream):
        pltpu.async_copy(h_hbm.at[idx_v.at[j]], rows_v, sem).wait()
        pltpu.sync_copy(
            rows_v, out_hbm.at[pl.ds(w * rows_per_w + j * n_stream, n_stream)])


def _make_gather(n, d, b):
    rows_per_w = b // NW
    n_stream = 128
    mesh = plsc.VectorSubcoreMesh(core_axis_name="c", subcore_axis_name="s")
    return pl.kernel(
        functools.partial(_gather_body, rows_per_w, n_stream),
        out_type=jax.ShapeDtypeStruct((b, d), jnp.float32),
        mesh=mesh,
        scratch_types=[
            pltpu.VMEM((rows_per_w // n_stream, n_stream), jnp.int32),
            pltpu.VMEM((n_stream, d), jnp.float32),
            pltpu.SemaphoreType.DMA,
        ],
    )


def _dot_t(a, w):
    return lax.dot_general(a, w, (((1,), (1,)), ((), ())),
                           preferred_element_type=jnp.float32)


def _dense_body(final, n, p_ref, c_ref, h_ref, wl_ref, wr_ref, b_ref,
                g_ref, bt_ref, o_ref):
    agg = p_ref[0, 0:n] + p_ref[1, 0:n]
    cnt = c_ref[0, 0:n, 0:1] + c_ref[1, 0:n, 0:1]
    mean = agg / jnp.maximum(cnt, 1.0)
    t = (_dot_t(mean, wl_ref[...]) + _dot_t(h_ref[...], wr_ref[...])
         + b_ref[...])
    mu = jnp.mean(t, axis=0, keepdims=True)
    var = jnp.mean((t - mu) ** 2, axis=0, keepdims=True)
    hn = g_ref[...] * (t - mu) * lax.rsqrt(var + 1e-5) + bt_ref[...]
    hn = jnp.where(hn >= 0, hn, 0.01 * hn)
    if final:
        hn = jnp.where(jnp.isnan(hn), jnp.float32(1e-14), hn)
        hn = jnp.clip(hn, _F32_MIN, _F32_MAX)
    o_ref[...] = hn


def _make_dense(n, h, final):
    return pl.pallas_call(
        functools.partial(_dense_body, final, n),
        out_shape=jax.ShapeDtypeStruct((n, h), jnp.float32),
    )


def _mlp_body(nc, rows_ref, feat_ref, w0u_ref, w0v_ref, w0f_ref, b0_ref,
              w1_ref, b1_ref, w2_ref, b2_ref, y_ref, p_ref):
    hu = rows_ref[0:nc, :]
    hv = rows_ref[nc:2 * nc, :]
    z = (_dot_t(hu, w0u_ref[...]) + _dot_t(hv, w0v_ref[...])
         + feat_ref[...] * w0f_ref[...] + b0_ref[...])
    z = jnp.where(z >= 0, z, 0.01 * z)
    z = _dot_t(z, w1_ref[...]) + b1_ref[...]
    z = jnp.where(z >= 0, z, 0.01 * z)
    y = jnp.sum(z * w2_ref[...], axis=1, keepdims=True) + b2_ref[0, 0]
    m = jnp.max(y)
    ey = jnp.exp(y - m)
    p_ref[...] = ey / jnp.sum(ey)
    y_ref[...] = y


def _make_mlp(nc):
    return pl.pallas_call(
        functools.partial(_mlp_body, nc),
        out_shape=(jax.ShapeDtypeStruct((nc, 1), jnp.float32),
                   jax.ShapeDtypeStruct((nc, 1), jnp.float32)),
    )


def kernel(x, edge_index, cand_uv, cand_feat,
           conv0_wl, conv0_wr, conv0_b,
           conv1_wl, conv1_wr, conv1_b,
           bn0_g, bn0_b, bn1_g, bn1_b,
           mlp_w0, mlp_b0, mlp_w1, mlp_b1, mlp_w2, mlp_b2):
    n, d = x.shape
    e = edge_index.shape[1]
    nc = cand_uv.shape[0]
    h = conv0_wl.shape[0]
    n_iter = e // (NW * CHUNK)

    npad = ((n + 8 * NSUB - 1) // (8 * NSUB)) * (8 * NSUB)
    src3 = edge_index[0].astype(jnp.int32).reshape(NW, n_iter, CHUNK)
    dst3 = edge_index[1].astype(jnp.int32).reshape(NW, n_iter, CHUNK)
    z128 = jnp.zeros((npad, d), jnp.float32)
    ones = jnp.ones((CHUNK, d), jnp.float32)
    cidx = jnp.concatenate(
        [cand_uv[:, 0], cand_uv[:, 1]]).astype(jnp.int32).reshape(NW, -1, 128)

    seg = _make_seg_sum(npad, n_iter, d)
    cntk = _make_cnt(npad, n_iter, d)
    gat = _make_gather(n, h, 2 * nc)
    dense0 = _make_dense(n, h, False)
    dense1 = _make_dense(n, h, True)
    mlp = _make_mlp(nc)

    cntp = cntk(dst3, z128, ones)
    aggp0 = seg(x, src3, dst3, z128)
    h1 = dense0(aggp0, cntp, x, conv0_wl, conv0_wr,
                conv0_b.reshape(1, h), bn0_g.reshape(1, h),
                bn0_b.reshape(1, h))
    aggp1 = seg(h1, src3, dst3, z128)
    h2 = dense1(aggp1, cntp, h1, conv1_wl, conv1_wr,
                conv1_b.reshape(1, h), bn1_g.reshape(1, h),
                bn1_b.reshape(1, h))
    rows = gat(h2, cidx)
    y, p = mlp(rows, cand_feat,
               mlp_w0[:, 0:d], mlp_w0[:, d:2 * d],
               mlp_w0[:, 2 * d].reshape(1, -1), mlp_b0.reshape(1, -1),
               mlp_w1, mlp_b1.reshape(1, -1),
               mlp_w2, mlp_b2.reshape(1, 1))
    return (y, p)


# speedup vs baseline: 2.6759x; 1.1312x over previous
import functools

import jax
import jax.numpy as jnp
from jax import lax
from jax.experimental import pallas as pl
from jax.experimental.pallas import tpu as pltpu
from jax.experimental.pallas import tpu_sc as plsc

NCORES = 2
NSUB = 16
NW = NCORES * NSUB
CHUNK = 125
NBUF = 4

_F32_MAX = float(jnp.finfo(jnp.float32).max)
_F32_MIN = float(jnp.finfo(jnp.float32).min)


def _seg_body(n_iter, rps, x_hbm, src_hbm, dst_hbm, zero_hbm,
              aggp_hbm, src_v, dst_v, rows_v, acc_sh, sem):
    c = lax.axis_index("c")
    s = lax.axis_index("s")
    w = s * NCORES + c
    pltpu.sync_copy(src_hbm.at[w], src_v)
    pltpu.sync_copy(dst_hbm.at[w], dst_v)
    r0 = s * rps
    pltpu.sync_copy(zero_hbm.at[pl.ds(r0, rps)], acc_sh.at[pl.ds(r0, rps)])
    plsc.subcore_barrier()

    def step(i, carry):
        pltpu.async_copy(x_hbm.at[src_v.at[i]], rows_v, sem).wait()
        pltpu.sync_copy(rows_v, acc_sh.at[dst_v.at[i]], add=True)
        return carry

    lax.fori_loop(0, n_iter, step, 0)
    plsc.subcore_barrier()
    pltpu.sync_copy(acc_sh.at[pl.ds(r0, rps)],
                    aggp_hbm.at[c].at[pl.ds(r0, rps)])


def _make_seg_sum(n, n_iter, d):
    rps = n // NSUB
    mesh = plsc.VectorSubcoreMesh(core_axis_name="c", subcore_axis_name="s")
    return pl.kernel(
        functools.partial(_seg_body, n_iter, rps),
        out_type=jax.ShapeDtypeStruct((NCORES, n, d), jnp.float32),
        mesh=mesh,
        scratch_types=[
            pltpu.VMEM((n_iter, CHUNK), jnp.int32),
            pltpu.VMEM((n_iter, CHUNK), jnp.int32),
            pltpu.VMEM((CHUNK, d), jnp.float32),
            pltpu.VMEM_SHARED((n, d), jnp.float32),
            pltpu.SemaphoreType.DMA,
        ],
    )


def _hist_body(n_g16, dst16_hbm, z1_hbm, hist_hbm, dst16_v, hist_v):
    c = lax.axis_index("c")
    s = lax.axis_index("s")
    w = s * NCORES + c
    pltpu.sync_copy(dst16_hbm.at[w], dst16_v)
    pltpu.sync_copy(z1_hbm, hist_v)
    ones16 = jnp.ones((16,), jnp.float32)

    def hstep(i, carry):
        plsc.addupdate_scatter(hist_v, [dst16_v[i]], ones16)
        return carry

    lax.fori_loop(0, n_g16, hstep, 0)
    pltpu.sync_copy(hist_v, hist_hbm.at[w])


def _make_hist(n, n_g16):
    mesh = plsc.VectorSubcoreMesh(core_axis_name="c", subcore_axis_name="s")
    return pl.kernel(
        functools.partial(_hist_body, n_g16),
        out_type=jax.ShapeDtypeStruct((NW, n), jnp.float32),
        mesh=mesh,
        compiler_params=pltpu.CompilerParams(needs_layout_passes=False),
        scratch_types=[
            pltpu.VMEM((n_g16, 16), jnp.int32),
            pltpu.VMEM((n,), jnp.float32),
        ],
    )


def _gather_body(rows_per_w, n_stream, h_hbm, idx_hbm, out_hbm,
                 idx_v, rows_v, sem):
    c = lax.axis_index("c")
    s = lax.axis_index("s")
    w = s * NCORES + c
    pltpu.sync_copy(idx_hbm.at[w], idx_v)
    for j in range(rows_per_w // n_stream):
        pltpu.async_copy(h_hbm.at[idx_v.at[j]], rows_v, sem).wait()
        pltpu.sync_copy(
            rows_v, out_hbm.at[pl.ds(w * rows_per_w + j * n_stream, n_stream)])


def _make_gather(n, d, b):
    rows_per_w = b // NW
    n_stream = 128
    mesh = plsc.VectorSubcoreMesh(core_axis_name="c", subcore_axis_name="s")
    return pl.kernel(
        functools.partial(_gather_body, rows_per_w, n_stream),
        out_type=jax.ShapeDtypeStruct((b, d), jnp.float32),
        mesh=mesh,
        scratch_types=[
            pltpu.VMEM((rows_per_w // n_stream, n_stream), jnp.int32),
            pltpu.VMEM((n_stream, d), jnp.float32),
            pltpu.SemaphoreType.DMA,
        ],
    )


def _dot_t(a, w):
    return lax.dot_general(a, w, (((1,), (1,)), ((), ())),
                           preferred_element_type=jnp.float32)


def _dense_body(final, n, p_ref, c_ref, h_ref, wl_ref, wr_ref, b_ref,
                g_ref, bt_ref, o_ref):
    agg = p_ref[0, 0:n] + p_ref[1, 0:n]
    cnt = jnp.transpose(jnp.sum(c_ref[...], axis=0, keepdims=True),
                        (1, 0))[0:n]
    mean = agg / jnp.maximum(cnt, 1.0)
    t = (_dot_t(mean, wl_ref[...]) + _dot_t(h_ref[...], wr_ref[...])
         + b_ref[...])
    mu = jnp.mean(t, axis=0, keepdims=True)
    var = jnp.mean((t - mu) ** 2, axis=0, keepdims=True)
    hn = g_ref[...] * (t - mu) * lax.rsqrt(var + 1e-5) + bt_ref[...]
    hn = jnp.where(hn >= 0, hn, 0.01 * hn)
    if final:
        hn = jnp.where(jnp.isnan(hn), jnp.float32(1e-14), hn)
        hn = jnp.clip(hn, _F32_MIN, _F32_MAX)
    o_ref[...] = hn


def _make_dense(n, h, final):
    return pl.pallas_call(
        functools.partial(_dense_body, final, n),
        out_shape=jax.ShapeDtypeStruct((n, h), jnp.float32),
    )


def _mlp_body(nc, rows_ref, feat_ref, w0u_ref, w0v_ref, w0f_ref, b0_ref,
              w1_ref, b1_ref, w2_ref, b2_ref, y_ref, p_ref):
    hu = rows_ref[0:nc, :]
    hv = rows_ref[nc:2 * nc, :]
    z = (_dot_t(hu, w0u_ref[...]) + _dot_t(hv, w0v_ref[...])
         + feat_ref[...] * w0f_ref[...] + b0_ref[...])
    z = jnp.where(z >= 0, z, 0.01 * z)
    z = _dot_t(z, w1_ref[...]) + b1_ref[...]
    z = jnp.where(z >= 0, z, 0.01 * z)
    y = jnp.sum(z * w2_ref[...], axis=1, keepdims=True) + b2_ref[0, 0]
    m = jnp.max(y)
    ey = jnp.exp(y - m)
    p_ref[...] = ey / jnp.sum(ey)
    y_ref[...] = y


def _make_mlp(nc):
    return pl.pallas_call(
        functools.partial(_mlp_body, nc),
        out_shape=(jax.ShapeDtypeStruct((nc, 1), jnp.float32),
                   jax.ShapeDtypeStruct((nc, 1), jnp.float32)),
    )


def kernel(x, edge_index, cand_uv, cand_feat,
           conv0_wl, conv0_wr, conv0_b,
           conv1_wl, conv1_wr, conv1_b,
           bn0_g, bn0_b, bn1_g, bn1_b,
           mlp_w0, mlp_b0, mlp_w1, mlp_b1, mlp_w2, mlp_b2):
    n, d = x.shape
    e = edge_index.shape[1]
    nc = cand_uv.shape[0]
    h = conv0_wl.shape[0]
    n_iter = e // (NW * CHUNK)
    n_g16 = e // (NW * 16)

    npad = ((n + 8 * NSUB - 1) // (8 * NSUB)) * (8 * NSUB)
    src3 = edge_index[0].astype(jnp.int32).reshape(NW, n_iter, CHUNK)
    dst3 = edge_index[1].astype(jnp.int32).reshape(NW, n_iter, CHUNK)
    dst16 = edge_index[1].astype(jnp.int32).reshape(NW, n_g16, 16)
    z128 = jnp.zeros((npad, d), jnp.float32)
    z1 = jnp.zeros((npad,), jnp.float32)
    cidx = jnp.concatenate(
        [cand_uv[:, 0], cand_uv[:, 1]]).astype(jnp.int32).reshape(NW, -1, 128)

    seg = _make_seg_sum(npad, n_iter, d)
    histk = _make_hist(npad, n_g16)
    gat = _make_gather(n, h, 2 * nc)
    dense0 = _make_dense(n, h, False)
    dense1 = _make_dense(n, h, True)
    mlp = _make_mlp(nc)

    cntp = histk(dst16, z1)
    aggp0 = seg(x, src3, dst3, z128)
    h1 = dense0(aggp0, cntp, x, conv0_wl, conv0_wr,
                conv0_b.reshape(1, h), bn0_g.reshape(1, h),
                bn0_b.reshape(1, h))
    aggp1 = seg(h1, src3, dst3, z128)
    h2 = dense1(aggp1, cntp, h1, conv1_wl, conv1_wr,
                conv1_b.reshape(1, h), bn1_g.reshape(1, h),
                bn1_b.reshape(1, h))
    rows = gat(h2, cidx)
    y, p = mlp(rows, cand_feat,
               mlp_w0[:, 0:d], mlp_w0[:, d:2 * d],
               mlp_w0[:, 2 * d].reshape(1, -1), mlp_b0.reshape(1, -1),
               mlp_w1, mlp_b1.reshape(1, -1),
               mlp_w2, mlp_b2.reshape(1, 1))
    return (y, p)


# speedup vs baseline: 2.6763x; 1.0001x over previous
import functools

import jax
import jax.numpy as jnp
from jax import lax
from jax.experimental import pallas as pl
from jax.experimental.pallas import tpu as pltpu
from jax.experimental.pallas import tpu_sc as plsc

NCORES = 2
NSUB = 16
NW = NCORES * NSUB
CHUNK = 125
NBUF = 4

_F32_MAX = float(jnp.finfo(jnp.float32).max)
_F32_MIN = float(jnp.finfo(jnp.float32).min)


def _seg_body(n_iter, rps, rows_d, x_hbm, src_hbm, dst_hbm, zero_hbm,
              aggp_hbm, src_v, dst_v, acc_sh):
    c = lax.axis_index("c")
    s = lax.axis_index("s")
    w = s * NCORES + c
    pltpu.sync_copy(src_hbm.at[w], src_v)
    pltpu.sync_copy(dst_hbm.at[w], dst_v)
    r0 = s * rps
    pltpu.sync_copy(zero_hbm.at[pl.ds(r0, rps)], acc_sh.at[pl.ds(r0, rps)])
    plsc.subcore_barrier()

    @plsc.parallel_loop(0, n_iter, 1, unroll=2)
    def step(i):
        def inner(rows, isem):
            pltpu.async_copy(x_hbm.at[src_v.at[i]], rows, isem).wait()
            pltpu.sync_copy(rows, acc_sh.at[dst_v.at[i]], add=True)

        pl.run_scoped(inner, pltpu.VMEM((CHUNK, rows_d), jnp.float32),
                      pltpu.SemaphoreType.DMA)

    plsc.subcore_barrier()
    pltpu.sync_copy(acc_sh.at[pl.ds(r0, rps)],
                    aggp_hbm.at[c].at[pl.ds(r0, rps)])


def _make_seg_sum(n, n_iter, d):
    rps = n // NSUB
    mesh = plsc.VectorSubcoreMesh(core_axis_name="c", subcore_axis_name="s")
    return pl.kernel(
        functools.partial(_seg_body, n_iter, rps, d),
        out_type=jax.ShapeDtypeStruct((NCORES, n, d), jnp.float32),
        mesh=mesh,
        scratch_types=[
            pltpu.VMEM((n_iter, CHUNK), jnp.int32),
            pltpu.VMEM((n_iter, CHUNK), jnp.int32),
            pltpu.VMEM_SHARED((n, d), jnp.float32),
        ],
    )


def _hist_body(n_g16, dst16_hbm, z1_hbm, hist_hbm, dst16_v, hist_v):
    c = lax.axis_index("c")
    s = lax.axis_index("s")
    w = s * NCORES + c
    pltpu.sync_copy(dst16_hbm.at[w], dst16_v)
    pltpu.sync_copy(z1_hbm, hist_v)
    ones16 = jnp.ones((16,), jnp.float32)

    def hstep(i, carry):
        plsc.addupdate_scatter(hist_v, [dst16_v[i]], ones16)
        return carry

    lax.fori_loop(0, n_g16, hstep, 0)
    pltpu.sync_copy(hist_v, hist_hbm.at[w])


def _make_hist(n, n_g16):
    mesh = plsc.VectorSubcoreMesh(core_axis_name="c", subcore_axis_name="s")
    return pl.kernel(
        functools.partial(_hist_body, n_g16),
        out_type=jax.ShapeDtypeStruct((NW, n), jnp.float32),
        mesh=mesh,
        compiler_params=pltpu.CompilerParams(needs_layout_passes=False),
        scratch_types=[
            pltpu.VMEM((n_g16, 16), jnp.int32),
            pltpu.VMEM((n,), jnp.float32),
        ],
    )


def _gather_body(rows_per_w, n_stream, h_hbm, idx_hbm, out_hbm,
                 idx_v, rows_v, sem):
    c = lax.axis_index("c")
    s = lax.axis_index("s")
    w = s * NCORES + c
    pltpu.sync_copy(idx_hbm.at[w], idx_v)
    for j in range(rows_per_w // n_stream):
        pltpu.async_copy(h_hbm.at[idx_v.at[j]], rows_v, sem).wait()
        pltpu.sync_copy(
            rows_v, out_hbm.at[pl.ds(w * rows_per_w + j * n_stream, n_stream)])


def _make_gather(n, d, b):
    rows_per_w = b // NW
    n_stream = 128
    mesh = plsc.VectorSubcoreMesh(core_axis_name="c", subcore_axis_name="s")
    return pl.kernel(
        functools.partial(_gather_body, rows_per_w, n_stream),
        out_type=jax.ShapeDtypeStruct((b, d), jnp.float32),
        mesh=mesh,
        scratch_types=[
            pltpu.VMEM((rows_per_w // n_stream, n_stream), jnp.int32),
            pltpu.VMEM((n_stream, d), jnp.float32),
            pltpu.SemaphoreType.DMA,
        ],
    )


def _dot_t(a, w):
    return lax.dot_general(a, w, (((1,), (1,)), ((), ())),
                           preferred_element_type=jnp.float32)


def _dense_body(final, n, p_ref, c_ref, h_ref, wl_ref, wr_ref, b_ref,
                g_ref, bt_ref, o_ref):
    agg = p_ref[0, 0:n] + p_ref[1, 0:n]
    cnt = jnp.transpose(jnp.sum(c_ref[...], axis=0, keepdims=True),
                        (1, 0))[0:n]
    mean = agg / jnp.maximum(cnt, 1.0)
    t = (_dot_t(mean, wl_ref[...]) + _dot_t(h_ref[...], wr_ref[...])
         + b_ref[...])
    mu = jnp.mean(t, axis=0, keepdims=True)
    var = jnp.mean((t - mu) ** 2, axis=0, keepdims=True)
    hn = g_ref[...] * (t - mu) * lax.rsqrt(var + 1e-5) + bt_ref[...]
    hn = jnp.where(hn >= 0, hn, 0.01 * hn)
    if final:
        hn = jnp.where(jnp.isnan(hn), jnp.float32(1e-14), hn)
        hn = jnp.clip(hn, _F32_MIN, _F32_MAX)
    o_ref[...] = hn


def _make_dense(n, h, final):
    return pl.pallas_call(
        functools.partial(_dense_body, final, n),
        out_shape=jax.ShapeDtypeStruct((n, h), jnp.float32),
    )


def _mlp_body(nc, rows_ref, feat_ref, w0u_ref, w0v_ref, w0f_ref, b0_ref,
              w1_ref, b1_ref, w2_ref, b2_ref, y_ref, p_ref):
    hu = rows_ref[0:nc, :]
    hv = rows_ref[nc:2 * nc, :]
    z = (_dot_t(hu, w0u_ref[...]) + _dot_t(hv, w0v_ref[...])
         + feat_ref[...] * w0f_ref[...] + b0_ref[...])
    z = jnp.where(z >= 0, z, 0.01 * z)
    z = _dot_t(z, w1_ref[...]) + b1_ref[...]
    z = jnp.where(z >= 0, z, 0.01 * z)
    y = jnp.sum(z * w2_ref[...], axis=1, keepdims=True) + b2_ref[0, 0]
    m = jnp.max(y)
    ey = jnp.exp(y - m)
    p_ref[...] = ey / jnp.sum(ey)
    y_ref[...] = y


def _make_mlp(nc):
    return pl.pallas_call(
        functools.partial(_mlp_body, nc),
        out_shape=(jax.ShapeDtypeStruct((nc, 1), jnp.float32),
                   jax.ShapeDtypeStruct((nc, 1), jnp.float32)),
    )


def kernel(x, edge_index, cand_uv, cand_feat,
           conv0_wl, conv0_wr, conv0_b,
           conv1_wl, conv1_wr, conv1_b,
           bn0_g, bn0_b, bn1_g, bn1_b,
           mlp_w0, mlp_b0, mlp_w1, mlp_b1, mlp_w2, mlp_b2):
    n, d = x.shape
    e = edge_index.shape[1]
    nc = cand_uv.shape[0]
    h = conv0_wl.shape[0]
    n_iter = e // (NW * CHUNK)
    n_g16 = e // (NW * 16)

    npad = ((n + 8 * NSUB - 1) // (8 * NSUB)) * (8 * NSUB)
    src3 = edge_index[0].astype(jnp.int32).reshape(NW, n_iter, CHUNK)
    dst3 = edge_index[1].astype(jnp.int32).reshape(NW, n_iter, CHUNK)
    dst16 = edge_index[1].astype(jnp.int32).reshape(NW, n_g16, 16)
    z128 = jnp.zeros((npad, d), jnp.float32)
    z1 = jnp.zeros((npad,), jnp.float32)
    cidx = jnp.concatenate(
        [cand_uv[:, 0], cand_uv[:, 1]]).astype(jnp.int32).reshape(NW, -1, 128)

    seg = _make_seg_sum(npad, n_iter, d)
    histk = _make_hist(npad, n_g16)
    gat = _make_gather(n, h, 2 * nc)
    dense0 = _make_dense(n, h, False)
    dense1 = _make_dense(n, h, True)
    mlp = _make_mlp(nc)

    cntp = histk(dst16, z1)
    aggp0 = seg(x, src3, dst3, z128)
    h1 = dense0(aggp0, cntp, x, conv0_wl, conv0_wr,
                conv0_b.reshape(1, h), bn0_g.reshape(1, h),
                bn0_b.reshape(1, h))
    aggp1 = seg(h1, src3, dst3, z128)
    h2 = dense1(aggp1, cntp, h1, conv1_wl, conv1_wr,
                conv1_b.reshape(1, h), bn1_g.reshape(1, h),
                bn1_b.reshape(1, h))
    rows = gat(h2, cidx)
    y, p = mlp(rows, cand_feat,
               mlp_w0[:, 0:d], mlp_w0[:, d:2 * d],
               mlp_w0[:, 2 * d].reshape(1, -1), mlp_b0.reshape(1, -1),
               mlp_w1, mlp_b1.reshape(1, -1),
               mlp_w2, mlp_b2.reshape(1, 1))
    return (y, p)
